# speedup vs baseline: 1.4737x; 1.4737x over previous
import functools

import jax
import jax.numpy as jnp
from jax import lax
from jax.experimental import pallas as pl
from jax.experimental.pallas import tpu as pltpu
from jax.experimental.pallas import tpu_sc as plsc

B = 800
V = 200
D = 98304
NCH = 32
CW = D // NCH
W = 8
NWIN = B // W
NBUF = 4
NADJ = B // 16

_mesh = plsc.VectorSubcoreMesh(core_axis_name="c", subcore_axis_name="s")


@functools.partial(
    pl.kernel,
    out_type=jax.ShapeDtypeStruct((B, D), jnp.float32),
    mesh=_mesh,
    scratch_types=[
        pltpu.VMEM((B,), jnp.int32),
        pltpu.VMEM((B,), jnp.int32),
        pltpu.VMEM((NBUF, W, CW), jnp.float32),
        pltpu.SemaphoreType.DMA((NBUF,)),
        pltpu.SemaphoreType.DMA,
    ],
)
def _sc_gather(idx_hbm, table_hbm, out_hbm, idx_raw, idx_adj, buf, gsem, ssem):
    cid = lax.axis_index("c")
    sid = lax.axis_index("s")
    wid = sid * 2 + cid
    col = pl.multiple_of(wid * CW, CW)

    pltpu.sync_copy(idx_hbm, idx_raw)

    def adj_body(i, carry):
        off = pl.multiple_of(i * 16, 16)
        idx_adj[pl.ds(off, 16)] = idx_raw[pl.ds(off, 16)] * NCH + wid
        return carry

    lax.fori_loop(0, NADJ, adj_body, 0)

    def gather(g, b):
        roff = pl.multiple_of(g * W, W)
        return pltpu.make_async_copy(
            table_hbm.at[idx_adj.at[pl.ds(roff, W)]], buf.at[b], gsem.at[b]
        )

    def scatter(g, b):
        roff = pl.multiple_of(g * W, W)
        return pltpu.make_async_copy(
            buf.at[b], out_hbm.at[pl.ds(roff, W), pl.ds(col, CW)], ssem
        )

    for b in range(NBUF):
        gather(b, b).start()

    def loop_body(gi, carry):
        for b in range(NBUF):
            g = gi * NBUF + b
            gather(g, b).wait()
            sc = scatter(g, b)
            sc.start()
            sc.wait()
            gather(g + NBUF, b).start()
        return carry

    lax.fori_loop(0, NWIN // NBUF - 1, loop_body, 0)

    for b in range(NBUF):
        g = NWIN - NBUF + b
        gather(g, b).wait()
        sc = scatter(g, b)
        sc.start()
        sc.wait()


def kernel(prefix_tokens, embedding):
    idx = prefix_tokens.reshape(-1).astype(jnp.int32)
    table_r = embedding.reshape(V * NCH, CW)
    out = _sc_gather(idx, table_r)
    return out.reshape(prefix_tokens.shape[0], prefix_tokens.shape[1], D)


# speedup vs baseline: 1.4772x; 1.0024x over previous
import functools

import jax
import jax.numpy as jnp
from jax import lax
from jax.experimental import pallas as pl
from jax.experimental.pallas import tpu as pltpu
from jax.experimental.pallas import tpu_sc as plsc

B = 800
V = 200
D = 98304
NCH = 32
CW = D // NCH
W = 8
NWIN = B // W
NBUF = 5
SDEPTH = 2
NADJ = B // 16

_mesh = plsc.VectorSubcoreMesh(core_axis_name="c", subcore_axis_name="s")


@functools.partial(
    pl.kernel,
    out_type=jax.ShapeDtypeStruct((B, D), jnp.float32),
    mesh=_mesh,
    scratch_types=[
        pltpu.VMEM((B,), jnp.int32),
        pltpu.VMEM((B,), jnp.int32),
        pltpu.VMEM((NBUF, W, CW), jnp.float32),
        pltpu.SemaphoreType.DMA((NBUF,)),
        pltpu.SemaphoreType.DMA((NBUF,)),
    ],
)
def _sc_gather(idx_hbm, table_hbm, out_hbm, idx_raw, idx_adj, buf, gsem, ssem):
    cid = lax.axis_index("c")
    sid = lax.axis_index("s")
    wid = sid * 2 + cid
    col = pl.multiple_of(wid * CW, CW)

    pltpu.sync_copy(idx_hbm, idx_raw)

    def adj_body(i, carry):
        off = pl.multiple_of(i * 16, 16)
        idx_adj[pl.ds(off, 16)] = idx_raw[pl.ds(off, 16)] * NCH + wid
        return carry

    lax.fori_loop(0, NADJ, adj_body, 0)

    def gather(g, b):
        roff = pl.multiple_of(g * W, W)
        return pltpu.make_async_copy(
            table_hbm.at[idx_adj.at[pl.ds(roff, W)]], buf.at[b], gsem.at[b]
        )

    def scatter(g, b):
        roff = pl.multiple_of(g * W, W)
        return pltpu.make_async_copy(
            buf.at[b], out_hbm.at[pl.ds(roff, W), pl.ds(col, CW)], ssem.at[b]
        )

    for b in range(NBUF):
        gather(b, b).start()

    for g in range(SDEPTH):
        b = g % NBUF
        gather(g, b).wait()
        scatter(g, b).start()

    NSTEADY = NWIN - NBUF
    assert NSTEADY % NBUF == 0

    def steady(k, carry):
        for j in range(NBUF):
            g = SDEPTH + k * NBUF + j
            b = (SDEPTH + j) % NBUF
            scatter(g - SDEPTH, j).wait()
            gather(g - SDEPTH + NBUF, j).start()
            gather(g, b).wait()
            scatter(g, b).start()
        return carry

    lax.fori_loop(0, NSTEADY // NBUF, steady, 0)

    for g in range(NWIN - NBUF + SDEPTH, NWIN):
        b = g % NBUF
        gather(g, b).wait()
        scatter(g, b).start()

    for g in range(NWIN - NBUF, NWIN):
        scatter(g, g % NBUF).wait()


def kernel(prefix_tokens, embedding):
    idx = prefix_tokens.reshape(-1).astype(jnp.int32)
    table_r = embedding.reshape(V * NCH, CW)
    out = _sc_gather(idx, table_r)
    return out.reshape(prefix_tokens.shape[0], prefix_tokens.shape[1], D)
